# baseline (speedup 1.0000x reference)
import functools

import jax
import jax.numpy as jnp
from jax import lax
from jax.experimental import pallas as pl
from jax.experimental.pallas import tpu as pltpu
from jax.experimental.pallas import tpu_sc as plsc

DIM = 64
BATCH = 16384
NC = 2
NS = 16
NW = NC * NS
BPW = BATCH // NW
CHUNK = 128
NCHUNK = BPW // CHUNK
LANES = 16
GROUPS = BPW // LANES


def _sc_body(ctab, xtab, cidx, xidx, out,
             cidx_v, xidx_v, crows, xrows, out_v, sem):
    wid = lax.axis_index("s") * NC + lax.axis_index("c")
    base_chunk = wid * NCHUNK

    pltpu.sync_copy(cidx.at[pl.ds(base_chunk, NCHUNK)], cidx_v)
    pltpu.sync_copy(xidx.at[pl.ds(base_chunk, NCHUNK)], xidx_v)

    copies = []
    for j in range(NCHUNK):
        copies.append(pltpu.async_copy(
            ctab.at[cidx_v.at[j]], crows.at[pl.ds(j * CHUNK, CHUNK)], sem))
        copies.append(pltpu.async_copy(
            xtab.at[xidx_v.at[j]], xrows.at[pl.ds(j * CHUNK, CHUNK)], sem))
    for c in copies:
        c.wait()

    iota = lax.iota(jnp.int32, LANES)

    def group(g, carry):
        vecs = []
        for r in range(LANES):
            row = g * LANES + r
            acc = crows[row, pl.ds(0, LANES)] * xrows[row, pl.ds(0, LANES)]
            for c in range(1, DIM // LANES):
                acc = acc + (crows[row, pl.ds(c * LANES, LANES)]
                             * xrows[row, pl.ds(c * LANES, LANES)])
            s = jnp.sum(acc)
            vecs.append(jnp.where(iota == r, s, jnp.float32(0)))
        while len(vecs) > 1:
            vecs = [a + b for a, b in zip(vecs[::2], vecs[1::2])]
        out_v[pl.ds(g * LANES, LANES)] = vecs[0]
        return carry

    lax.fori_loop(0, GROUPS, group, None)
    pltpu.sync_copy(out_v, out.at[pl.ds(wid * BPW, BPW)])


@jax.jit
def kernel(center_words, context_words, center_table, context_table):
    cidx = center_words.astype(jnp.int32).reshape(NW * NCHUNK, CHUNK)
    xidx = context_words.astype(jnp.int32).reshape(NW * NCHUNK, CHUNK)
    f = pl.kernel(
        _sc_body,
        mesh=plsc.VectorSubcoreMesh(core_axis_name="c", subcore_axis_name="s"),
        out_type=jax.ShapeDtypeStruct((BATCH,), jnp.float32),
        compiler_params=pltpu.CompilerParams(
            needs_layout_passes=False, use_tc_tiling_on_sc=False),
        scratch_types=[
            pltpu.VMEM((NCHUNK, CHUNK), jnp.int32),
            pltpu.VMEM((NCHUNK, CHUNK), jnp.int32),
            pltpu.VMEM((BPW, DIM), jnp.float32),
            pltpu.VMEM((BPW, DIM), jnp.float32),
            pltpu.VMEM((BPW,), jnp.float32),
            pltpu.SemaphoreType.DMA,
        ],
    )
    return f(center_table, context_table, cidx, xidx)


# speedup vs baseline: 1.5811x; 1.5811x over previous
import functools

import jax
import jax.numpy as jnp
from jax import lax
from jax.experimental import pallas as pl
from jax.experimental.pallas import tpu as pltpu
from jax.experimental.pallas import tpu_sc as plsc

VOCAB = 100000
DIM = 64
BATCH = 16384
NC = 2
NS = 16
NW = NC * NS
LANES = 16
CHUNK = 2048
NCHUNK = BATCH // CHUNK
BPW = BATCH // NW

_params = pltpu.CompilerParams(
    needs_layout_passes=False, use_tc_tiling_on_sc=True)
_mesh = plsc.VectorSubcoreMesh(core_axis_name="c", subcore_axis_name="s")


def _gather_body(ctab, xtab, cidx, xidx, gc, gx,
                 row_v, idx_v, buf_v, sem):
    wid = lax.axis_index("s") * NC + lax.axis_index("c")

    def do_feature(tab, out, j):
        pltpu.sync_copy(tab.at[j], row_v)

        def group(gi, carry):
            ids = idx_v[pl.ds(gi * LANES, LANES)]
            buf_v[pl.ds((gi % (CHUNK // LANES)) * LANES, LANES)] = (
                plsc.load_gather(row_v, [ids]))
            return carry

        for k in range(NCHUNK):
            lax.fori_loop(k * (CHUNK // LANES), (k + 1) * (CHUNK // LANES),
                          group, None)
            pltpu.sync_copy(buf_v, out.at[j, pl.ds(k * CHUNK, CHUNK)])

    pltpu.sync_copy(cidx, idx_v)
    do_feature(ctab, gc, wid)
    do_feature(ctab, gc, wid + NW)
    pltpu.sync_copy(xidx, idx_v)
    do_feature(xtab, gx, wid)
    do_feature(xtab, gx, wid + NW)


def _dot_body(gc, gx, out, gc_v, gx_v, out_v, sem):
    wid = lax.axis_index("s") * NC + lax.axis_index("c")
    base = wid * BPW
    pltpu.sync_copy(gc.at[:, pl.ds(base, BPW)], gc_v)
    pltpu.sync_copy(gx.at[:, pl.ds(base, BPW)], gx_v)

    def group(g, carry):
        vecs = [gc_v[j, pl.ds(g * LANES, LANES)]
                * gx_v[j, pl.ds(g * LANES, LANES)]
                for j in range(DIM)]
        while len(vecs) > 1:
            vecs = [a + b for a, b in zip(vecs[::2], vecs[1::2])]
        out_v[pl.ds(g * LANES, LANES)] = vecs[0]
        return carry

    lax.fori_loop(0, BPW // LANES, group, None)
    pltpu.sync_copy(out_v, out.at[pl.ds(base, BPW)])


@jax.jit
def kernel(center_words, context_words, center_table, context_table):
    cidx = center_words.astype(jnp.int32)
    xidx = context_words.astype(jnp.int32)
    ctab_t = center_table.T
    xtab_t = context_table.T

    gather = pl.kernel(
        _gather_body,
        mesh=_mesh,
        out_type=(
            jax.ShapeDtypeStruct((DIM, BATCH), jnp.float32),
            jax.ShapeDtypeStruct((DIM, BATCH), jnp.float32),
        ),
        scratch_types=[
            pltpu.VMEM((VOCAB,), jnp.float32),
            pltpu.VMEM((BATCH,), jnp.int32),
            pltpu.VMEM((CHUNK,), jnp.float32),
            pltpu.SemaphoreType.DMA,
        ],
        compiler_params=_params,
    )
    gc, gx = gather(ctab_t, xtab_t, cidx, xidx)

    dot = pl.kernel(
        _dot_body,
        mesh=_mesh,
        out_type=jax.ShapeDtypeStruct((BATCH,), jnp.float32),
        scratch_types=[
            pltpu.VMEM((DIM, BPW), jnp.float32),
            pltpu.VMEM((DIM, BPW), jnp.float32),
            pltpu.VMEM((BPW,), jnp.float32),
            pltpu.SemaphoreType.DMA,
        ],
        compiler_params=_params,
    )
    return dot(gc, gx)
